# baseline (speedup 1.0000x reference)
import functools

import jax
import jax.numpy as jnp
from jax import lax
from jax.experimental import pallas as pl
from jax.experimental.pallas import tpu as pltpu
from jax.experimental.pallas import tpu_sc as plsc

_NUM_CORES = 2
_NUM_SUBCORES = 16
_NUM_WORKERS = _NUM_CORES * _NUM_SUBCORES


def _make_gather(B, D, idx_dtype):
    b_per_w = B // _NUM_WORKERS
    mesh = plsc.VectorSubcoreMesh(core_axis_name="c", subcore_axis_name="s")

    @functools.partial(
        pl.kernel,
        mesh=mesh,
        out_type=(
            jax.ShapeDtypeStruct((B, D), jnp.float32),
            jax.ShapeDtypeStruct((B, D), jnp.float32),
        ),
        scratch_types=[
            pltpu.VMEM((b_per_w,), idx_dtype),
            pltpu.VMEM((b_per_w, D), jnp.float32),
            pltpu.VMEM((b_per_w,), idx_dtype),
            pltpu.VMEM((b_per_w, D), jnp.float32),
            pltpu.SemaphoreType.DMA,
            pltpu.SemaphoreType.DMA,
        ],
        compiler_params=pltpu.CompilerParams(use_tc_tiling_on_sc=False),
    )
    def gather_kernel(uidx_hbm, iidx_hbm, utab_hbm, itab_hbm,
                      uout_hbm, iout_hbm,
                      uidx_v, urows_v, iidx_v, irows_v, usem, isem):
        wid = lax.axis_index("s") * _NUM_CORES + lax.axis_index("c")
        base = wid * b_per_w
        pltpu.sync_copy(uidx_hbm.at[pl.ds(base, b_per_w)], uidx_v)
        pltpu.sync_copy(iidx_hbm.at[pl.ds(base, b_per_w)], iidx_v)
        ucp = pltpu.async_copy(utab_hbm.at[uidx_v], urows_v, usem)
        icp = pltpu.async_copy(itab_hbm.at[iidx_v], irows_v, isem)
        ucp.wait()
        pltpu.sync_copy(urows_v, uout_hbm.at[pl.ds(base, b_per_w)])
        icp.wait()
        pltpu.sync_copy(irows_v, iout_hbm.at[pl.ds(base, b_per_w)])

    return gather_kernel


def _mlp_body(u_ref, v_ref, w1a_ref, w1b_ref, b1_ref, w2_ref, b2_ref,
              w3_ref, b3_ref, w4_ref, b4_ref, out_ref):
    dot = functools.partial(jnp.dot, preferred_element_type=jnp.float32,
                            precision=lax.Precision.HIGHEST)
    x = dot(u_ref[...], w1a_ref[...]) + dot(v_ref[...], w1b_ref[...])
    x = jnp.maximum(x + b1_ref[...], 0.0)
    x = jnp.maximum(dot(x, w2_ref[...]) + b2_ref[...], 0.0)
    x = jnp.maximum(dot(x, w3_ref[...]) + b3_ref[...], 0.0)
    logits = jnp.sum(x * w4_ref[...], axis=1) + b4_ref[0]
    out_ref[...] = jax.nn.sigmoid(logits)


def _make_mlp(B, D, blk):
    grid = (B // blk,)
    full = lambda shape: pl.BlockSpec(shape, lambda i: (0,) * len(shape))
    return pl.pallas_call(
        _mlp_body,
        grid=grid,
        in_specs=[
            pl.BlockSpec((blk, D), lambda i: (i, 0)),
            pl.BlockSpec((blk, D), lambda i: (i, 0)),
            full((D, 64)), full((D, 64)), full((1, 64)),
            full((64, 32)), full((1, 32)),
            full((32, 16)), full((1, 16)),
            full((1, 16)), full((1,)),
        ],
        out_specs=pl.BlockSpec((blk,), lambda i: (i,)),
        out_shape=jax.ShapeDtypeStruct((B,), jnp.float32),
    )


def kernel(user_indices, item_indices, user_table, item_table,
           W1, b1, W2, b2, W3, b3, W4, b4):
    B = user_indices.shape[0]
    D = user_table.shape[1]
    user_vec, item_vec = _make_gather(B, D, user_indices.dtype)(
        user_indices, item_indices, user_table, item_table)
    w1a = W1[:, :D].T
    w1b = W1[:, D:].T
    mlp = _make_mlp(B, D, 2048)
    return mlp(user_vec, item_vec,
               w1a, w1b, b1.reshape(1, 64),
               W2.T, b2.reshape(1, 32),
               W3.T, b3.reshape(1, 16),
               W4.reshape(1, 16), b4)


# speedup vs baseline: 3.0953x; 3.0953x over previous
import functools

import jax
import jax.numpy as jnp
from jax import lax
from jax.experimental import pallas as pl
from jax.experimental.pallas import tpu as pltpu
from jax.experimental.pallas import tpu_sc as plsc

_NUM_CORES = 2
_NUM_SUBCORES = 16
_NUM_WORKERS = _NUM_CORES * _NUM_SUBCORES
_NBUF = 8


def _make_gather(B, D, T):
    bpw = B // _NUM_WORKERS
    lanes = 128
    mesh = plsc.VectorSubcoreMesh(core_axis_name="c", subcore_axis_name="s")

    @functools.partial(
        pl.kernel,
        mesh=mesh,
        out_type=(
            jax.ShapeDtypeStruct((B * D,), jnp.float32),
            jax.ShapeDtypeStruct((B * D,), jnp.float32),
        ),
        scratch_types=[
            pltpu.VMEM((B,), jnp.int32),
            pltpu.VMEM((B,), jnp.int32),
            pltpu.VMEM((_NBUF, D, lanes), jnp.float32),
            pltpu.VMEM((bpw * D,), jnp.float32),
            pltpu.VMEM((bpw * D,), jnp.float32),
            pltpu.SemaphoreType.DMA((_NBUF,)),
            pltpu.SemaphoreType.DMA,
        ],
        compiler_params=pltpu.CompilerParams(use_tc_tiling_on_sc=True,
                                             needs_layout_passes=False),
    )
    def gather_kernel(uidx_hbm, iidx_hbm, utT_hbm, itT_hbm,
                      uout_hbm, iout_hbm,
                      uidx_v, iidx_v, stage_v, urows_v, irows_v, sems, sem0):
        wid = lax.axis_index("s") * _NUM_CORES + lax.axis_index("c")
        base = wid * bpw
        pltpu.sync_copy(uidx_hbm, uidx_v)
        pltpu.sync_copy(iidx_hbm, iidx_v)
        jrow = lax.iota(jnp.int32, 16)
        jhi = jrow + 16

        def run_table(tabT_hbm, idx_v, rows_v):
            def fetch(slot, tc):
                pltpu.async_copy(
                    tabT_hbm.at[:, pl.ds(pl.multiple_of(tc * lanes, lanes),
                                         lanes)],
                    stage_v.at[slot], sems.at[slot])

            v0 = idx_v[pl.ds(base, 16)]
            for l in range(_NBUF):
                fetch(l, v0[l] >> 7)

            def chunk(c, _):
                v = idx_v[pl.ds(base + c * 16, 16)]
                nxt = jnp.minimum(base + c * 16 + _NBUF, B - 16)
                vn = idx_v[pl.ds(nxt, 16)]
                for l in range(16):
                    slot = l % _NBUF
                    r = c * 16 + l
                    pltpu.make_async_copy(
                        tabT_hbm.at[:, pl.ds(0, lanes)],
                        stage_v.at[slot], sems.at[slot]).wait()
                    i = v[l]
                    col = jnp.broadcast_to(i & (lanes - 1), (16,))
                    g0 = plsc.load_gather(stage_v.at[slot], [jrow, col])
                    g1 = plsc.load_gather(stage_v.at[slot], [jhi, col])
                    rows_v[pl.ds(r * D, 16)] = g0
                    rows_v[pl.ds(r * D + 16, 16)] = g1
                    fetch(slot, vn[l] >> 7)
                return _

            lax.fori_loop(0, bpw // 16, chunk, 0)
            for slot in range(_NBUF):
                pltpu.make_async_copy(
                    tabT_hbm.at[:, pl.ds(0, lanes)],
                    stage_v.at[slot], sems.at[slot]).wait()

        run_table(utT_hbm, uidx_v, urows_v)
        pltpu.async_copy(urows_v, uout_hbm.at[pl.ds(base * D, bpw * D)],
                         sem0)
        run_table(itT_hbm, iidx_v, irows_v)
        pltpu.sync_copy(irows_v, iout_hbm.at[pl.ds(base * D, bpw * D)])
        pltpu.make_async_copy(uout_hbm.at[pl.ds(base * D, bpw * D)],
                              urows_v, sem0).wait()

    return gather_kernel


def _mlp_body(u_ref, v_ref, w1a_ref, w1b_ref, b1_ref, w2_ref, b2_ref,
              w3_ref, b3_ref, w4_ref, b4_ref, out_ref):
    dot = functools.partial(jnp.dot, preferred_element_type=jnp.float32,
                            precision=lax.Precision.HIGHEST)
    x = dot(u_ref[...], w1a_ref[...]) + dot(v_ref[...], w1b_ref[...])
    x = jnp.maximum(x + b1_ref[...], 0.0)
    x = jnp.maximum(dot(x, w2_ref[...]) + b2_ref[...], 0.0)
    x = jnp.maximum(dot(x, w3_ref[...]) + b3_ref[...], 0.0)
    logits = jnp.sum(x * w4_ref[...], axis=1) + b4_ref[0]
    out_ref[...] = jax.nn.sigmoid(logits)


def _make_mlp(B, D, blk):
    grid = (B // blk,)
    full = lambda shape: pl.BlockSpec(shape, lambda i: (0,) * len(shape))
    return pl.pallas_call(
        _mlp_body,
        grid=grid,
        in_specs=[
            pl.BlockSpec((blk, D), lambda i: (i, 0)),
            pl.BlockSpec((blk, D), lambda i: (i, 0)),
            full((D, 64)), full((D, 64)), full((1, 64)),
            full((64, 32)), full((1, 32)),
            full((32, 16)), full((1, 16)),
            full((1, 16)), full((1,)),
        ],
        out_specs=pl.BlockSpec((blk,), lambda i: (i,)),
        out_shape=jax.ShapeDtypeStruct((B,), jnp.float32),
    )


def kernel(user_indices, item_indices, user_table, item_table,
           W1, b1, W2, b2, W3, b3, W4, b4):
    B = user_indices.shape[0]
    D = user_table.shape[1]
    T = user_table.shape[0]
    uflat, iflat = _make_gather(B, D, T)(
        user_indices, item_indices, user_table.T, item_table.T)
    user_vec = uflat.reshape(B, D)
    item_vec = iflat.reshape(B, D)
    w1a = W1[:, :D].T
    w1b = W1[:, D:].T
    mlp = _make_mlp(B, D, 2048)
    return mlp(user_vec, item_vec,
               w1a, w1b, b1.reshape(1, 64),
               W2.T, b2.reshape(1, 32),
               W3.T, b3.reshape(1, 16),
               W4.reshape(1, 16), b4)
